# baseline (speedup 1.0000x reference)
import jax
import jax.numpy as jnp
from jax.experimental import pallas as pl

C = 256
H = 100
W = 100
Z = 8
C_BLK = 32


def _body(col_t_ref, row_t_ref, z_t_ref, out_ref):
    col_t = col_t_ref[...]
    row_t = row_t_ref[...]
    z_t = z_t_ref[...]
    rz = (row_t[:, :, None] + z_t[:, None, :]).reshape(C_BLK, 1, W * Z)
    out_ref[...] = col_t[:, :, None] + rz


def kernel(row_weight, col_weight, z_weight, bs, h, w, z):
    col_t = col_weight.T
    row_t = row_weight.T
    z_t = z_weight.T
    out3 = pl.pallas_call(
        _body,
        grid=(C // C_BLK,),
        in_specs=[
            pl.BlockSpec((C_BLK, H), lambda i: (i, 0)),
            pl.BlockSpec((C_BLK, W), lambda i: (i, 0)),
            pl.BlockSpec((C_BLK, Z), lambda i: (i, 0)),
        ],
        out_specs=pl.BlockSpec((C_BLK, H, W * Z), lambda i: (i, 0, 0)),
        out_shape=jax.ShapeDtypeStruct((C, H, W * Z), jnp.float32),
    )(col_t, row_t, z_t)
    return out3.reshape(1, C, H, W, Z)


# speedup vs baseline: 1.0850x; 1.0850x over previous
import jax
import jax.numpy as jnp
from jax.experimental import pallas as pl

C = 256
H = 100
W = 100
Z = 8
C_BLK = 32


def _body(col_t_ref, row_rep_ref, z_tile_ref, out_ref):
    rz = row_rep_ref[...] + z_tile_ref[...]
    out_ref[...] = col_t_ref[...][:, :, None] + rz[:, None, :]


def kernel(row_weight, col_weight, z_weight, bs, h, w, z):
    col_t = col_weight.T
    row_rep = jnp.repeat(row_weight.T, Z, axis=1)
    z_tile = jnp.tile(z_weight.T, (1, W))
    out3 = pl.pallas_call(
        _body,
        grid=(C // C_BLK,),
        in_specs=[
            pl.BlockSpec((C_BLK, H), lambda i: (i, 0)),
            pl.BlockSpec((C_BLK, W * Z), lambda i: (i, 0)),
            pl.BlockSpec((C_BLK, W * Z), lambda i: (i, 0)),
        ],
        out_specs=pl.BlockSpec((C_BLK, H, W * Z), lambda i: (i, 0, 0)),
        out_shape=jax.ShapeDtypeStruct((C, H, W * Z), jnp.float32),
    )(col_t, row_rep, z_tile)
    return out3.reshape(1, C, H, W, Z)
